# baseline (speedup 1.0000x reference)
import jax
import jax.numpy as jnp
from jax import lax
from jax.experimental import pallas as pl

_B = 41
_BR = 2048


def _tc_body(v_ref, d_ref, o_ref):
    o_ref[...] = jnp.zeros((_BR, _B), jnp.float32) + v_ref[0]


def kernel(scalar):
    n = scalar.shape[0]
    dummy = jnp.zeros((n, _B), jnp.float32)
    return pl.pallas_call(
        _tc_body,
        grid=(n // _BR,),
        in_specs=[pl.BlockSpec((_BR,), lambda i: (i,)),
                  pl.BlockSpec(memory_space=pl.ANY)],
        out_specs=pl.BlockSpec((_BR, _B), lambda i: (i, 0)),
        out_shape=jax.ShapeDtypeStruct((n, _B), jnp.float32),
        input_output_aliases={1: 0},
    )(scalar, dummy)


# speedup vs baseline: 1.2885x; 1.2885x over previous
import jax
import jax.numpy as jnp
from jax import lax
from jax.experimental import pallas as pl
from jax.experimental.pallas import tpu as pltpu
from jax.experimental.pallas import tpu_sc as plsc

_R = 20
_B = 2 * _R + 1
_NC = 2
_NS = 16
_NW = _NC * _NS
_L = 16

_C = 256
_NV = _C // _L
_BR = 2048

_SC_FRAC_NUM = 88
_SC_UNITS = 128

_LN2 = 0.6931471805599453
_SQRT2 = 1.4142135623730951


def _symlog_twohot(x):
    a = jnp.abs(x) + 1.0
    bits = lax.bitcast_convert_type(a, jnp.int32)
    e = lax.shift_right_logical(bits, 23) - 127
    mbits = lax.bitwise_or(lax.bitwise_and(bits, 0x007FFFFF), 0x3F800000)
    m = lax.bitcast_convert_type(mbits, jnp.float32)
    big = m > _SQRT2
    m = jnp.where(big, m * 0.5, m)
    e = e + jnp.where(big, 1, 0)
    z = (m - 1.0) / (m + 1.0)
    z2 = z * z
    lnm = 2.0 * z * (1.0 + z2 * (1.0 / 3.0 + z2 * (0.2 + z2 * (1.0 / 7.0))))
    val = jnp.sign(x) * (e.astype(jnp.float32) * _LN2 + lnm)
    val = jnp.minimum(jnp.maximum(val, -float(_R)), float(_R))
    ti = val.astype(jnp.int32)
    tf = ti.astype(jnp.float32)
    neg = val < tf
    fl_f = jnp.where(neg, tf - 1.0, tf)
    fl_i = jnp.where(neg, ti - 1, ti)
    prob = val - fl_f
    lo = fl_i + _R
    up = lo + 1
    w_up = jnp.where(up < _B, prob, 0.0)
    up_c = jnp.minimum(up, _B - 1)
    return lo, up_c, 1.0 - prob, w_up


def _sc_body(x_hbm, out_hbm, x_v, buf0, buf1, slo0, sup0, slo1, sup1,
             sem_in, sem0, sem1):
    rows_w = x_v.shape[0]
    chunks = rows_w // _C
    wid = lax.axis_index("s") * _NC + lax.axis_index("c")
    base = wid * rows_w

    in_cp = pltpu.async_copy(x_hbm.at[pl.ds(base, rows_w)], x_v, sem_in)

    zf = jnp.zeros((_L,), jnp.float32)
    zi = jnp.zeros((_L,), jnp.int32)
    iota = lax.iota(jnp.int32, _L)
    bufs = (buf0, buf1)
    slos = (slo0, slo1)
    sups = (sup0, sup1)
    sems = (sem0, sem1)

    @plsc.parallel_loop(0, _NV, unroll=4)
    def _(j):
        slo0[pl.ds(j * _L, _L)] = zi
        sup0[pl.ds(j * _L, _L)] = zi
        slo1[pl.ds(j * _L, _L)] = zi
        sup1[pl.ds(j * _L, _L)] = zi

    @plsc.parallel_loop(0, _NV)
    def _(j):
        rows = j * _L + iota
        for col in range(_B):
            cols = jnp.full((_L,), col, jnp.int32)
            plsc.store_scatter(buf0, [rows, cols], zf)
            plsc.store_scatter(buf1, [rows, cols], zf)

    in_cp.wait()

    def process(i, b, do_wait):
        buf, slo, sup, sem = bufs[b], slos[b], sups[b], sems[b]
        if do_wait:
            pltpu.make_async_copy(
                buf, out_hbm.at[pl.ds(base + (i - 2) * _C, _C)], sem).wait()

        @plsc.parallel_loop(0, _NV, unroll=4)
        def _(j):
            rows = j * _L + iota
            plsc.store_scatter(buf, [rows, slo[pl.ds(j * _L, _L)]], zf)
            plsc.store_scatter(buf, [rows, sup[pl.ds(j * _L, _L)]], zf)
            x = x_v[pl.ds(i * _C + j * _L, _L)]
            lo, up, w_lo, w_up = _symlog_twohot(x)
            plsc.store_scatter(buf, [rows, up], w_up)
            plsc.store_scatter(buf, [rows, lo], w_lo)
            slo[pl.ds(j * _L, _L)] = lo
            sup[pl.ds(j * _L, _L)] = up

        pltpu.async_copy(buf, out_hbm.at[pl.ds(base + i * _C, _C)], sem)

    process(0, 0, False)
    process(1, 1, False)

    def outer(i2, c):
        process(i2 * 2, 0, True)
        process(i2 * 2 + 1, 1, True)
        return c

    lax.fori_loop(1, chunks // 2, outer, 0)

    for b, last in ((0, chunks - 2), (1, chunks - 1)):
        pltpu.make_async_copy(
            bufs[b], out_hbm.at[pl.ds(base + last * _C, _C)], sems[b]).wait()


def _tc_expand_body(x_ref, o_ref):
    x = x_ref[...]
    val = jnp.sign(x) * jnp.log(jnp.abs(x) + 1.0)
    u = jnp.minimum(jnp.maximum(val, -float(_R)), float(_R)) + float(_R)
    cols = lax.broadcasted_iota(jnp.int32, (_BR, _B), 1).astype(jnp.float32)
    o_ref[...] = jnp.maximum(1.0 - jnp.abs(u[:, None] - cols), 0.0)


def kernel(scalar):
    n = scalar.shape[0]
    m = n // _SC_UNITS * _SC_FRAC_NUM
    rows_w = m // _NW
    mesh = plsc.VectorSubcoreMesh(core_axis_name="c", subcore_axis_name="s")
    sc_part = pl.kernel(
        _sc_body,
        out_type=jax.ShapeDtypeStruct((m, _B), jnp.float32),
        mesh=mesh,
        compiler_params=pltpu.CompilerParams(
            needs_layout_passes=False, use_tc_tiling_on_sc=True),
        scratch_types=[
            pltpu.VMEM((rows_w,), jnp.float32),
            pltpu.VMEM((_C, _B), jnp.float32),
            pltpu.VMEM((_C, _B), jnp.float32),
            pltpu.VMEM((_C,), jnp.int32),
            pltpu.VMEM((_C,), jnp.int32),
            pltpu.VMEM((_C,), jnp.int32),
            pltpu.VMEM((_C,), jnp.int32),
            pltpu.SemaphoreType.DMA,
            pltpu.SemaphoreType.DMA,
            pltpu.SemaphoreType.DMA,
        ],
    )(scalar)

    off = m // _BR
    tc_part = pl.pallas_call(
        _tc_expand_body,
        grid=((n - m) // _BR,),
        in_specs=[pl.BlockSpec((_BR,), lambda i: (i + off,))],
        out_specs=pl.BlockSpec((_BR, _B), lambda i: (i, 0)),
        out_shape=jax.ShapeDtypeStruct((n - m, _B), jnp.float32),
    )(scalar)

    return jnp.concatenate([sc_part, tc_part], axis=0)


# speedup vs baseline: 1.5884x; 1.2327x over previous
import jax
import jax.numpy as jnp
from jax import lax
from jax.experimental import pallas as pl
from jax.experimental.pallas import tpu as pltpu
from jax.experimental.pallas import tpu_sc as plsc

_R = 20
_B = 2 * _R + 1
_NC = 2
_NS = 16
_NW = _NC * _NS
_L = 16

_C = 256
_NV = _C // _L

_LN2 = 0.6931471805599453
_SQRT2 = 1.4142135623730951


def _symlog_twohot(x):
    a = jnp.abs(x) + 1.0
    bits = lax.bitcast_convert_type(a, jnp.int32)
    e = lax.shift_right_logical(bits, 23) - 127
    mbits = lax.bitwise_or(lax.bitwise_and(bits, 0x007FFFFF), 0x3F800000)
    m = lax.bitcast_convert_type(mbits, jnp.float32)
    big = m > _SQRT2
    m = jnp.where(big, m * 0.5, m)
    e = e + jnp.where(big, 1, 0)
    z = (m - 1.0) / (m + 1.0)
    z2 = z * z
    lnm = 2.0 * z * (1.0 + z2 * (1.0 / 3.0 + z2 * (0.2 + z2 * (1.0 / 7.0))))
    lna = e.astype(jnp.float32) * _LN2 + lnm
    sgn = lax.bitwise_and(lax.bitcast_convert_type(x, jnp.int32),
                          jnp.int32(-2147483648))
    val = lax.bitcast_convert_type(
        lax.bitwise_or(lax.bitcast_convert_type(lna, jnp.int32), sgn),
        jnp.float32)
    val = jnp.minimum(jnp.maximum(val, -float(_R)), float(_R))
    ti = val.astype(jnp.int32)
    tf = ti.astype(jnp.float32)
    neg = val < tf
    fl_f = jnp.where(neg, tf - 1.0, tf)
    fl_i = jnp.where(neg, ti - 1, ti)
    prob = val - fl_f
    lo = fl_i + _R
    up = lo + 1
    w_up = jnp.where(up < _B, prob, 0.0)
    up_c = jnp.minimum(up, _B - 1)
    return lo, up_c, 1.0 - prob, w_up


def _sc_body(x_hbm, out_hbm, x_v, buf0, buf1, slo0, sup0, slo1, sup1,
             sem_in, sem0, sem1):
    rows_w = x_v.shape[0]
    chunks = rows_w // _C
    wid = lax.axis_index("s") * _NC + lax.axis_index("c")
    base = wid * rows_w

    in_cp = pltpu.async_copy(x_hbm.at[pl.ds(base, rows_w)], x_v, sem_in)

    zf = jnp.zeros((_L,), jnp.float32)
    zi = jnp.zeros((_L,), jnp.int32)
    iota = lax.iota(jnp.int32, _L)
    bufs = (buf0, buf1)
    slos = (slo0, slo1)
    sups = (sup0, sup1)
    sems = (sem0, sem1)

    @plsc.parallel_loop(0, _NV, unroll=4)
    def _(j):
        slo0[pl.ds(j * _L, _L)] = zi
        sup0[pl.ds(j * _L, _L)] = zi
        slo1[pl.ds(j * _L, _L)] = zi
        sup1[pl.ds(j * _L, _L)] = zi

    @plsc.parallel_loop(0, _NV)
    def _(j):
        rows = j * _L + iota
        for col in range(_B):
            cols = jnp.full((_L,), col, jnp.int32)
            plsc.store_scatter(buf0, [rows, cols], zf)
            plsc.store_scatter(buf1, [rows, cols], zf)

    in_cp.wait()

    def process(i, b, do_wait):
        buf, slo, sup, sem = bufs[b], slos[b], sups[b], sems[b]
        if do_wait:
            pltpu.make_async_copy(
                buf, out_hbm.at[pl.ds(base + (i - 2) * _C, _C)], sem).wait()

        @plsc.parallel_loop(0, _NV, unroll=8)
        def _(j):
            rows = j * _L + iota
            plsc.store_scatter(buf, [rows, slo[pl.ds(j * _L, _L)]], zf)
            plsc.store_scatter(buf, [rows, sup[pl.ds(j * _L, _L)]], zf)
            x = x_v[pl.ds(i * _C + j * _L, _L)]
            lo, up, w_lo, w_up = _symlog_twohot(x)
            plsc.store_scatter(buf, [rows, up], w_up)
            plsc.store_scatter(buf, [rows, lo], w_lo)
            slo[pl.ds(j * _L, _L)] = lo
            sup[pl.ds(j * _L, _L)] = up

        pltpu.async_copy(buf, out_hbm.at[pl.ds(base + i * _C, _C)], sem)

    process(0, 0, False)
    process(1, 1, False)

    def outer(i2, c):
        process(i2 * 2, 0, True)
        process(i2 * 2 + 1, 1, True)
        return c

    lax.fori_loop(1, chunks // 2, outer, 0)

    for b, last in ((0, chunks - 2), (1, chunks - 1)):
        pltpu.make_async_copy(
            bufs[b], out_hbm.at[pl.ds(base + last * _C, _C)], sems[b]).wait()


def kernel(scalar):
    n = scalar.shape[0]
    rows_w = n // _NW
    mesh = plsc.VectorSubcoreMesh(core_axis_name="c", subcore_axis_name="s")
    f = pl.kernel(
        _sc_body,
        out_type=jax.ShapeDtypeStruct((n, _B), jnp.float32),
        mesh=mesh,
        compiler_params=pltpu.CompilerParams(
            needs_layout_passes=False, use_tc_tiling_on_sc=True),
        scratch_types=[
            pltpu.VMEM((rows_w,), jnp.float32),
            pltpu.VMEM((_C, _B), jnp.float32),
            pltpu.VMEM((_C, _B), jnp.float32),
            pltpu.VMEM((_C,), jnp.int32),
            pltpu.VMEM((_C,), jnp.int32),
            pltpu.VMEM((_C,), jnp.int32),
            pltpu.VMEM((_C,), jnp.int32),
            pltpu.SemaphoreType.DMA,
            pltpu.SemaphoreType.DMA,
            pltpu.SemaphoreType.DMA,
        ],
    )
    return f(scalar)
